# Initial kernel scaffold; baseline (speedup 1.0000x reference)
import jax
import jax.numpy as jnp
from jax.experimental import pallas as pl


def kernel(x, embeddings):
    raise NotImplementedError("write your pallas kernel here")


# speedup vs baseline: 1.2021x; 1.2021x over previous
import functools

import jax
import jax.numpy as jnp
from jax import lax
from jax.experimental import pallas as pl
from jax.experimental.pallas import tpu as pltpu
from jax.experimental.pallas import tpu_sc as plsc

K = 8192
D = 256
BN = 256


WINDOWS = (2816, 2816, 2560)


def _bf16_round_f32(a):
    u = lax.bitcast_convert_type(a, jnp.uint32)
    r = (u + jnp.uint32(0x7FFF) + ((u >> 16) & jnp.uint32(1))) \
        & jnp.uint32(0xFFFF0000)
    return lax.bitcast_convert_type(r, jnp.float32)


def _argmin_body(xx_ref, x_ref, emb_ref, yy_ref, out_ref):
    xy = lax.dot_general(
        x_ref[...], emb_ref[...],
        (((1,), (1,)), ((), ())),
        preferred_element_type=jnp.float32,
    )
    dist = (xx_ref[0] + yy_ref[...]) - 2.0 * xy
    best_v = jnp.full((BN,), jnp.inf, jnp.float32)
    best_i = jnp.zeros((BN,), jnp.int32)
    lo = 0
    for w in WINDOWS:
        seg = dist[:, lo:lo + w]
        wv = jnp.min(seg, axis=1)
        iota = lax.broadcasted_iota(jnp.int32, (BN, w), 1)
        wi = lo + jnp.min(jnp.where(seg == wv[:, None], iota, K), axis=1)
        new_wins = wv < best_v
        best_i = jnp.where(new_wins, wi, best_i)
        best_v = _bf16_round_f32(
            jnp.where(new_wins | (wv == best_v), wv, best_v))
        lo += w
    out_ref[...] = best_i.reshape(1, BN, 1)


def _compute_codes(xx, flat, embt, yy):
    n = flat.shape[0]
    nb = n // BN
    codes = pl.pallas_call(
        _argmin_body,
        grid=(nb,),
        in_specs=[
            pl.BlockSpec((1, BN, 1), lambda i: (i, 0, 0)),
            pl.BlockSpec((BN, D), lambda i: (i, 0)),
            pl.BlockSpec((K, D), lambda i: (0, 0)),
            pl.BlockSpec((1, K), lambda i: (0, 0)),
        ],
        out_specs=pl.BlockSpec((1, BN, 1), lambda i: (i, 0, 0)),
        out_shape=jax.ShapeDtypeStruct((nb, BN, 1), jnp.int32),
        compiler_params=pltpu.CompilerParams(
            dimension_semantics=("arbitrary",),
        ),
    )(xx.reshape(nb, BN, 1), flat.astype(jnp.bfloat16),
      embt.astype(jnp.bfloat16), yy)
    return codes.reshape(n)


CHUNK = 128


def _make_gather(n):
    info = plsc.get_sparse_core_info()
    nc, ns = info.num_cores, info.num_subcores
    nw = nc * ns
    b_per_w = n // nw
    n_chunks = b_per_w // CHUNK
    mesh = plsc.VectorSubcoreMesh(core_axis_name="c", subcore_axis_name="s")

    @functools.partial(
        pl.kernel,
        mesh=mesh,
        out_type=jax.ShapeDtypeStruct((n, D), jnp.float32),
        scratch_types=[
            pltpu.VMEM((CHUNK,), jnp.int32),
            pltpu.VMEM((CHUNK, D), jnp.float32),
            pltpu.SemaphoreType.DMA,
        ],
    )
    def gather_rows(emb_hbm, codes_hbm, out_hbm, idx_v, rows_v, sem):
        wid = lax.axis_index("s") * nc + lax.axis_index("c")
        for c in range(n_chunks):
            base = wid * b_per_w + c * CHUNK
            pltpu.sync_copy(codes_hbm.at[pl.ds(base, CHUNK)], idx_v)
            pltpu.async_copy(emb_hbm.at[idx_v], rows_v, sem).wait()
            pltpu.sync_copy(rows_v, out_hbm.at[pl.ds(base, CHUNK)])

    return gather_rows


def kernel(x, embeddings):
    input_shape = x.shape
    flat = x.reshape(-1, D)
    n = flat.shape[0]
    xx = jnp.sum(flat ** 2, axis=1, keepdims=True)
    yy = jnp.sum(embeddings ** 2, axis=1)
    codes = _compute_codes(xx, flat, embeddings, yy.reshape(1, K))
    qx = _make_gather(n)(embeddings, codes).reshape(input_shape)
    return x + lax.stop_gradient(qx - x)


# speedup vs baseline: 1.2817x; 1.0662x over previous
import functools

import jax
import jax.numpy as jnp
from jax import lax
from jax.experimental import pallas as pl
from jax.experimental.pallas import tpu as pltpu
from jax.experimental.pallas import tpu_sc as plsc

K = 8192
D = 256
BN = 512


WINDOWS = (2816, 2816, 2560)


def _bf16_round_f32(a):
    u = lax.bitcast_convert_type(a, jnp.uint32)
    r = (u + jnp.uint32(0x7FFF) + ((u >> 16) & jnp.uint32(1))) \
        & jnp.uint32(0xFFFF0000)
    return lax.bitcast_convert_type(r, jnp.float32)


def _argmin_body(xx_ref, x_ref, emb_ref, yy_ref, out_ref):
    xy = lax.dot_general(
        x_ref[...], emb_ref[...],
        (((1,), (1,)), ((), ())),
        preferred_element_type=jnp.float32,
    )
    dist = (xx_ref[0] + yy_ref[...]) - 2.0 * xy
    best_v = jnp.full((BN,), jnp.inf, jnp.float32)
    best_i = jnp.zeros((BN,), jnp.int32)
    lo = 0
    for w in WINDOWS:
        seg = dist[:, lo:lo + w]
        wv = jnp.min(seg, axis=1)
        iota = lax.broadcasted_iota(jnp.int32, (BN, w), 1)
        wi = lo + jnp.min(jnp.where(seg == wv[:, None], iota, K), axis=1)
        new_wins = wv < best_v
        best_i = jnp.where(new_wins, wi, best_i)
        best_v = _bf16_round_f32(
            jnp.where(new_wins | (wv == best_v), wv, best_v))
        lo += w
    out_ref[...] = best_i.reshape(1, BN, 1)


def _compute_codes(xx, flat, embt, yy):
    n = flat.shape[0]
    nb = n // BN
    codes = pl.pallas_call(
        _argmin_body,
        grid=(nb,),
        in_specs=[
            pl.BlockSpec((1, BN, 1), lambda i: (i, 0, 0)),
            pl.BlockSpec((BN, D), lambda i: (i, 0)),
            pl.BlockSpec((K, D), lambda i: (0, 0)),
            pl.BlockSpec((1, K), lambda i: (0, 0)),
        ],
        out_specs=pl.BlockSpec((1, BN, 1), lambda i: (i, 0, 0)),
        out_shape=jax.ShapeDtypeStruct((nb, BN, 1), jnp.int32),
        compiler_params=pltpu.CompilerParams(
            dimension_semantics=("arbitrary",),
        ),
    )(xx.reshape(nb, BN, 1), flat.astype(jnp.bfloat16),
      embt.astype(jnp.bfloat16), yy)
    return codes.reshape(n)


CHUNK = 128


def _make_gather(n):
    info = plsc.get_sparse_core_info()
    nc, ns = info.num_cores, info.num_subcores
    nw = nc * ns
    b_per_w = n // nw
    n_chunks = b_per_w // CHUNK
    mesh = plsc.VectorSubcoreMesh(core_axis_name="c", subcore_axis_name="s")

    @functools.partial(
        pl.kernel,
        mesh=mesh,
        out_type=jax.ShapeDtypeStruct((n, D), jnp.float32),
        scratch_types=[
            pltpu.VMEM((CHUNK,), jnp.int32),
            pltpu.VMEM((CHUNK, D), jnp.float32),
            pltpu.SemaphoreType.DMA,
        ],
    )
    def gather_rows(emb_hbm, codes_hbm, out_hbm, idx_v, rows_v, sem):
        wid = lax.axis_index("s") * nc + lax.axis_index("c")
        for c in range(n_chunks):
            base = wid * b_per_w + c * CHUNK
            pltpu.sync_copy(codes_hbm.at[pl.ds(base, CHUNK)], idx_v)
            pltpu.async_copy(emb_hbm.at[idx_v], rows_v, sem).wait()
            pltpu.sync_copy(rows_v, out_hbm.at[pl.ds(base, CHUNK)])

    return gather_rows


def kernel(x, embeddings):
    input_shape = x.shape
    flat = x.reshape(-1, D)
    n = flat.shape[0]
    xx = jnp.sum(flat ** 2, axis=1, keepdims=True)
    yy = jnp.sum(embeddings ** 2, axis=1)
    codes = _compute_codes(xx, flat, embeddings, yy.reshape(1, K))
    qx = _make_gather(n)(embeddings, codes).reshape(input_shape)
    return x + lax.stop_gradient(qx - x)


# speedup vs baseline: 1.2950x; 1.0104x over previous
import functools

import jax
import jax.numpy as jnp
from jax import lax
from jax.experimental import pallas as pl
from jax.experimental.pallas import tpu as pltpu
from jax.experimental.pallas import tpu_sc as plsc

K = 8192
D = 256
BN = 512


WINDOWS = (2816, 2816, 2560)


def _bf16_round_f32(a):
    u = lax.bitcast_convert_type(a, jnp.uint32)
    r = (u + jnp.uint32(0x7FFF) + ((u >> 16) & jnp.uint32(1))) \
        & jnp.uint32(0xFFFF0000)
    return lax.bitcast_convert_type(r, jnp.float32)


def _argmin_body(xx_ref, x_ref, emb_ref, yy_ref, out_ref):
    xb = x_ref[...]
    best_v = jnp.full((BN,), jnp.inf, jnp.float32)
    best_i = jnp.zeros((BN,), jnp.int32)
    lo = 0
    for w in WINDOWS:
        xy = lax.dot_general(
            xb, emb_ref[lo:lo + w, :],
            (((1,), (1,)), ((), ())),
            preferred_element_type=jnp.float32,
        )
        seg = (xx_ref[0] + yy_ref[:, lo:lo + w]) + xy
        wv = jnp.min(seg, axis=1)
        iota = lax.broadcasted_iota(jnp.int32, (BN, w), 1)
        wi = lo + jnp.min(jnp.where(seg == wv[:, None], iota, K), axis=1)
        new_wins = wv < best_v
        best_i = jnp.where(new_wins, wi, best_i)
        best_v = _bf16_round_f32(
            jnp.where(new_wins | (wv == best_v), wv, best_v))
        lo += w
    out_ref[...] = best_i.reshape(1, BN, 1)


def _compute_codes(xx, flat, embt, yy):
    n = flat.shape[0]
    nb = n // BN
    codes = pl.pallas_call(
        _argmin_body,
        grid=(nb,),
        in_specs=[
            pl.BlockSpec((1, BN, 1), lambda i: (i, 0, 0)),
            pl.BlockSpec((BN, D), lambda i: (i, 0)),
            pl.BlockSpec((K, D), lambda i: (0, 0)),
            pl.BlockSpec((1, K), lambda i: (0, 0)),
        ],
        out_specs=pl.BlockSpec((1, BN, 1), lambda i: (i, 0, 0)),
        out_shape=jax.ShapeDtypeStruct((nb, BN, 1), jnp.int32),
        compiler_params=pltpu.CompilerParams(
            dimension_semantics=("arbitrary",),
        ),
    )(xx.reshape(nb, BN, 1), (flat * -2.0).astype(jnp.bfloat16),
      embt.astype(jnp.bfloat16), yy)
    return codes.reshape(n)


CHUNK = 128


def _make_gather(n):
    info = plsc.get_sparse_core_info()
    nc, ns = info.num_cores, info.num_subcores
    nw = nc * ns
    b_per_w = n // nw
    n_chunks = b_per_w // CHUNK
    mesh = plsc.VectorSubcoreMesh(core_axis_name="c", subcore_axis_name="s")

    @functools.partial(
        pl.kernel,
        mesh=mesh,
        out_type=jax.ShapeDtypeStruct((n, D), jnp.float32),
        scratch_types=[
            pltpu.VMEM((CHUNK,), jnp.int32),
            pltpu.VMEM((CHUNK, D), jnp.float32),
            pltpu.SemaphoreType.DMA,
        ],
    )
    def gather_rows(emb_hbm, codes_hbm, out_hbm, idx_v, rows_v, sem):
        wid = lax.axis_index("s") * nc + lax.axis_index("c")
        for c in range(n_chunks):
            base = wid * b_per_w + c * CHUNK
            pltpu.sync_copy(codes_hbm.at[pl.ds(base, CHUNK)], idx_v)
            pltpu.async_copy(emb_hbm.at[idx_v], rows_v, sem).wait()
            pltpu.sync_copy(rows_v, out_hbm.at[pl.ds(base, CHUNK)])

    return gather_rows


def kernel(x, embeddings):
    input_shape = x.shape
    flat = x.reshape(-1, D)
    n = flat.shape[0]
    xx = jnp.sum(flat ** 2, axis=1, keepdims=True)
    yy = jnp.sum(embeddings ** 2, axis=1)
    codes = _compute_codes(xx, flat, embeddings, yy.reshape(1, K))
    qx = _make_gather(n)(embeddings, codes).reshape(input_shape)
    return x + lax.stop_gradient(qx - x)
